# baseline (device time: 686496 ns/iter reference)
import jax
import jax.numpy as jnp
from jax import lax
from jax.experimental import pallas as pl
from jax.experimental.pallas import tpu as pltpu

N_DEV = 8


def kernel(O, Wo):
    B, S, H, D = O.shape
    n_out = Wo.shape[1]
    s_per = S // N_DEV

    K = H * D
    Wb = Wo.astype(jnp.bfloat16)

    n_split = 4
    n_half = n_out // n_split

    def body(o_ref, w_ref, out_ref, comm_ref, stage_ref, x_ref, xr_ref,
             send_sems, recv_sems, load_sems):
        my = lax.axis_index("i")
        left = (my + N_DEV - 1) % N_DEV
        right = (my + 1) % N_DEV

        barrier_sem = pltpu.get_barrier_semaphore()
        for nbr in (left, right):
            pl.semaphore_signal(barrier_sem, inc=1, device_id=(nbr,),
                                device_id_type=pl.DeviceIdType.MESH,)
        pl.semaphore_wait(barrier_sem, 2)

        def load_chunk(c, slot):
            cp = pltpu.make_async_copy(
                o_ref.at[:, pl.ds(c * s_per, s_per), :, :],
                x_ref.at[slot],
                load_sems.at[slot],
            )
            cp.start()
            return cp

        def load_xr(slot):
            for b in range(B):
                xr_ref[b] = x_ref[slot, b].astype(jnp.bfloat16).reshape(
                    s_per, K)

        def partial(b, col_lo=0, col_n=n_out):
            return jnp.dot(xr_ref[b],
                           w_ref[:, pl.ds(col_lo, col_n)],
                           preferred_element_type=jnp.float32)

        def make_rdma(t, h, send_slot, recv_slot):
            sl = pl.ds(h * n_half, n_half)
            return pltpu.make_async_remote_copy(
                src_ref=comm_ref.at[send_slot, :, :, sl],
                dst_ref=comm_ref.at[recv_slot, :, :, sl],
                send_sem=send_sems.at[t, h],
                recv_sem=recv_sems.at[t, h],
                device_id=(right,),
                device_id_type=pl.DeviceIdType.MESH,
            )

        def compute_stage():
            for b in range(B):
                stage_ref[b] = partial(b).astype(jnp.bfloat16)

        c0 = (my + N_DEV - 1) % N_DEV
        cp0 = load_chunk(c0, 0)
        cp1 = load_chunk((my + N_DEV - 2) % N_DEV, 1)
        cp0.wait()
        load_xr(0)
        rdmas = [None] * n_split
        for h in range(n_split):
            sl = pl.ds(h * n_half, n_half)
            for b in range(B):
                comm_ref[0, b, :, sl] = partial(
                    b, h * n_half, n_half).astype(jnp.bfloat16)
            rdmas[h] = make_rdma(0, h, 0, 1)
            rdmas[h].start()
        cp1.wait()
        load_xr(1)
        compute_stage()

        for t in range(N_DEV - 1):
            send_slot = t % 2
            recv_slot = (t + 1) % 2
            cp = None
            if t < N_DEV - 2:
                cp = load_chunk((my + 2 * N_DEV - t - 3) % N_DEV, t % 2)
            for h in range(n_split):
                sl = pl.ds(h * n_half, n_half)
                rdmas[h].wait()
                if t < N_DEV - 2:
                    comm_ref[recv_slot, :, :, sl] = (
                        comm_ref[recv_slot, :, :, sl].astype(jnp.float32)
                        + stage_ref[:, :, sl].astype(jnp.float32)
                    ).astype(jnp.bfloat16)
                    rdmas[h] = make_rdma(t + 1, h, recv_slot, send_slot)
                    rdmas[h].start()
                else:
                    out_ref[:, :, sl] = (
                        comm_ref[recv_slot, :, :, sl].astype(jnp.float32)
                        + stage_ref[:, :, sl].astype(jnp.float32)
                    ).astype(jnp.bfloat16)
            if t < N_DEV - 2:
                cp.wait()
                load_xr(t % 2)
                compute_stage()

    out = pl.pallas_call(
        body,
        out_shape=jax.ShapeDtypeStruct((B, s_per, n_out), jnp.bfloat16),
        in_specs=[pl.BlockSpec(memory_space=pl.ANY),
                  pl.BlockSpec(memory_space=pltpu.VMEM)],
        out_specs=pl.BlockSpec(memory_space=pltpu.VMEM),
        scratch_shapes=[
            pltpu.VMEM((2, B, s_per, n_out), jnp.bfloat16),
            pltpu.VMEM((B, s_per, n_out), jnp.bfloat16),
            pltpu.VMEM((2, B, s_per, H, D), jnp.float32),
            pltpu.VMEM((B, s_per, K), jnp.bfloat16),
            pltpu.SemaphoreType.DMA((N_DEV - 1, n_split)),
            pltpu.SemaphoreType.DMA((N_DEV - 1, n_split)),
            pltpu.SemaphoreType.DMA((2,)),
        ],
        compiler_params=pltpu.CompilerParams(
            collective_id=0,
            vmem_limit_bytes=100 * 1024 * 1024,
        ),
    )(O, Wb)
    return out.astype(jnp.float32)


# device time: 672129 ns/iter; 1.0214x vs baseline; 1.0214x over previous
import jax
import jax.numpy as jnp
from jax import lax
from jax.experimental import pallas as pl
from jax.experimental.pallas import tpu as pltpu

N_DEV = 8


def kernel(O, Wo):
    B, S, H, D = O.shape
    K = H * D
    n_out = Wo.shape[1]
    s_per = S // N_DEV

    Ob = O.reshape(B, S, K).astype(jnp.bfloat16)
    Wb = Wo.astype(jnp.bfloat16)

    n_split = 4
    n_half = n_out // n_split

    def body(o_ref, w_ref, out_ref, comm_ref, stage_ref, x_ref,
             send_sems, recv_sems, load_sems):
        my = lax.axis_index("i")
        left = (my + N_DEV - 1) % N_DEV
        right = (my + 1) % N_DEV

        barrier_sem = pltpu.get_barrier_semaphore()
        for nbr in (left, right):
            pl.semaphore_signal(barrier_sem, inc=1, device_id=(nbr,),
                                device_id_type=pl.DeviceIdType.MESH)
        pl.semaphore_wait(barrier_sem, 2)

        def load_chunk(c, slot):
            cp = pltpu.make_async_copy(
                o_ref.at[:, pl.ds(c * s_per, s_per), :],
                x_ref.at[slot],
                load_sems.at[slot],
            )
            cp.start()
            return cp

        def partial(slot, b, col_lo=0, col_n=n_out):
            return jnp.dot(x_ref[slot, b],
                           w_ref[:, pl.ds(col_lo, col_n)],
                           preferred_element_type=jnp.float32)

        def make_rdma(t, h, send_slot, recv_slot):
            sl = pl.ds(h * n_half, n_half)
            return pltpu.make_async_remote_copy(
                src_ref=comm_ref.at[send_slot, :, :, sl],
                dst_ref=comm_ref.at[recv_slot, :, :, sl],
                send_sem=send_sems.at[t, h],
                recv_sem=recv_sems.at[t, h],
                device_id=(right,),
                device_id_type=pl.DeviceIdType.MESH,
            )

        def compute_stage(slot):
            for b in range(B):
                stage_ref[b] = partial(slot, b).astype(jnp.bfloat16)

        c0 = (my + N_DEV - 1) % N_DEV
        cp0 = load_chunk(c0, 0)
        cp1 = load_chunk((my + N_DEV - 2) % N_DEV, 1)
        cp0.wait()
        rdmas = [None] * n_split
        for h in range(n_split):
            sl = pl.ds(h * n_half, n_half)
            for b in range(B):
                comm_ref[0, b, :, sl] = partial(
                    0, b, h * n_half, n_half).astype(jnp.bfloat16)
            rdmas[h] = make_rdma(0, h, 0, 1)
            rdmas[h].start()
        cp1.wait()
        compute_stage(1)

        for t in range(N_DEV - 1):
            send_slot = t % 2
            recv_slot = (t + 1) % 2
            cp = None
            if t < N_DEV - 2:
                cp = load_chunk((my + 2 * N_DEV - t - 3) % N_DEV, t % 2)
            for h in range(n_split):
                sl = pl.ds(h * n_half, n_half)
                rdmas[h].wait()
                if t < N_DEV - 2:
                    comm_ref[recv_slot, :, :, sl] = (
                        comm_ref[recv_slot, :, :, sl].astype(jnp.float32)
                        + stage_ref[:, :, sl].astype(jnp.float32)
                    ).astype(jnp.bfloat16)
                    rdmas[h] = make_rdma(t + 1, h, recv_slot, send_slot)
                    rdmas[h].start()
                else:
                    out_ref[:, :, sl] = (
                        comm_ref[recv_slot, :, :, sl].astype(jnp.float32)
                        + stage_ref[:, :, sl].astype(jnp.float32)
                    ).astype(jnp.bfloat16)
            if t < N_DEV - 2:
                cp.wait()
                compute_stage(t % 2)

    return pl.pallas_call(
        body,
        out_shape=jax.ShapeDtypeStruct((B, s_per, n_out), jnp.bfloat16),
        in_specs=[pl.BlockSpec(memory_space=pl.ANY),
                  pl.BlockSpec(memory_space=pltpu.VMEM)],
        out_specs=pl.BlockSpec(memory_space=pltpu.VMEM),
        scratch_shapes=[
            pltpu.VMEM((2, B, s_per, n_out), jnp.bfloat16),
            pltpu.VMEM((B, s_per, n_out), jnp.bfloat16),
            pltpu.VMEM((2, B, s_per, K), jnp.bfloat16),
            pltpu.SemaphoreType.DMA((N_DEV - 1, n_split)),
            pltpu.SemaphoreType.DMA((N_DEV - 1, n_split)),
            pltpu.SemaphoreType.DMA((2,)),
        ],
        compiler_params=pltpu.CompilerParams(
            collective_id=0,
            vmem_limit_bytes=100 * 1024 * 1024,
        ),
    )(Ob, Wb)


# device time: 377094 ns/iter; 1.8205x vs baseline; 1.7824x over previous
import jax
import jax.numpy as jnp
from jax import lax
from jax.experimental import pallas as pl
from jax.experimental.pallas import tpu as pltpu

N_DEV = 8


def kernel(O, Wo):
    B, S, H, D = O.shape
    K = H * D
    n_out = Wo.shape[1]
    s_per = S // N_DEV
    n_half = n_out // 2

    Ob = O.reshape(B, S, K).astype(jnp.bfloat16)
    Wb = Wo.astype(jnp.bfloat16)

    def body(o_ref, w_ref, out_ref, comm_cw, comm_ccw, stage_cw, stage_ccw,
             x_ref, send_cw_sems, recv_cw_sems, send_ccw_sems, recv_ccw_sems,
             load_sems):
        my = lax.axis_index("i")
        left = (my + N_DEV - 1) % N_DEV
        right = (my + 1) % N_DEV

        barrier_sem = pltpu.get_barrier_semaphore()
        for nbr in (left, right):
            pl.semaphore_signal(barrier_sem, inc=1, device_id=(nbr,),
                                device_id_type=pl.DeviceIdType.MESH)
        pl.semaphore_wait(barrier_sem, 2)

        def load_chunk(c, slot):
            cp = pltpu.make_async_copy(
                o_ref.at[:, pl.ds(c * s_per, s_per), :],
                x_ref.at[slot],
                load_sems.at[slot],
            )
            cp.start()
            return cp

        def partial(slot, b, col_lo):
            return jnp.dot(x_ref[slot, b],
                           w_ref[:, pl.ds(col_lo, n_half)],
                           preferred_element_type=jnp.float32)

        dirs = [
            dict(sgn=-1, nbr=right, base=0, xbase=0, comm=comm_cw,
                 stage=stage_cw, send=send_cw_sems, recv=recv_cw_sems),
            dict(sgn=1, nbr=left, base=n_half, xbase=2, comm=comm_ccw,
                 stage=stage_ccw, send=send_ccw_sems, recv=recv_ccw_sems),
        ]

        def make_rdma(d, t, send_slot, recv_slot):
            return pltpu.make_async_remote_copy(
                src_ref=d["comm"].at[send_slot],
                dst_ref=d["comm"].at[recv_slot],
                send_sem=d["send"].at[t],
                recv_sem=d["recv"].at[t],
                device_id=(d["nbr"],),
                device_id_type=pl.DeviceIdType.MESH,
            )

        for d in dirs:
            d["cp0"] = load_chunk((my + N_DEV + d["sgn"]) % N_DEV, d["xbase"])
            d["cp1"] = load_chunk((my + N_DEV + 2 * d["sgn"]) % N_DEV,
                                  d["xbase"] + 1)
        for d in dirs:
            d["cp0"].wait()
            for b in range(B):
                d["comm"][0, b] = partial(
                    d["xbase"], b, d["base"]).astype(jnp.bfloat16)
            d["rdma"] = make_rdma(d, 0, 0, 1)
            d["rdma"].start()
        for d in dirs:
            d["cp1"].wait()
            for b in range(B):
                d["stage"][b] = partial(
                    d["xbase"] + 1, b, d["base"]).astype(jnp.bfloat16)

        for t in range(N_DEV - 1):
            send_slot = t % 2
            recv_slot = (t + 1) % 2
            if t < N_DEV - 2:
                for d in dirs:
                    d["cp"] = load_chunk(
                        (my + 3 * N_DEV + d["sgn"] * (t + 3)) % N_DEV,
                        d["xbase"] + t % 2)
            for d in dirs:
                d["rdma"].wait()
                if t < N_DEV - 2:
                    d["comm"][recv_slot] = (
                        d["comm"][recv_slot].astype(jnp.float32)
                        + d["stage"][...].astype(jnp.float32)
                    ).astype(jnp.bfloat16)
                    d["rdma"] = make_rdma(d, t + 1, recv_slot, send_slot)
                    d["rdma"].start()
                else:
                    out_ref[:, :, pl.ds(d["base"], n_half)] = (
                        d["comm"][recv_slot].astype(jnp.float32)
                        + d["stage"][...].astype(jnp.float32)
                    ).astype(jnp.bfloat16)
            if t < N_DEV - 2:
                for d in dirs:
                    d["cp"].wait()
                    for b in range(B):
                        d["stage"][b] = partial(
                            d["xbase"] + t % 2, b,
                            d["base"]).astype(jnp.bfloat16)

    return pl.pallas_call(
        body,
        out_shape=jax.ShapeDtypeStruct((B, s_per, n_out), jnp.bfloat16),
        in_specs=[pl.BlockSpec(memory_space=pl.ANY),
                  pl.BlockSpec(memory_space=pltpu.VMEM)],
        out_specs=pl.BlockSpec(memory_space=pltpu.VMEM),
        scratch_shapes=[
            pltpu.VMEM((2, B, s_per, n_half), jnp.bfloat16),
            pltpu.VMEM((2, B, s_per, n_half), jnp.bfloat16),
            pltpu.VMEM((B, s_per, n_half), jnp.bfloat16),
            pltpu.VMEM((B, s_per, n_half), jnp.bfloat16),
            pltpu.VMEM((4, B, s_per, K), jnp.bfloat16),
            pltpu.SemaphoreType.DMA((N_DEV - 1,)),
            pltpu.SemaphoreType.DMA((N_DEV - 1,)),
            pltpu.SemaphoreType.DMA((N_DEV - 1,)),
            pltpu.SemaphoreType.DMA((N_DEV - 1,)),
            pltpu.SemaphoreType.DMA((4,)),
        ],
        compiler_params=pltpu.CompilerParams(
            collective_id=0,
            vmem_limit_bytes=100 * 1024 * 1024,
        ),
    )(Ob, Wb)


# device time: 359559 ns/iter; 1.9093x vs baseline; 1.0488x over previous
import jax
import jax.numpy as jnp
from jax import lax
from jax.experimental import pallas as pl
from jax.experimental.pallas import tpu as pltpu

N_DEV = 8


def kernel(O, Wo):
    B, S, H, D = O.shape
    K = H * D
    n_out = Wo.shape[1]
    s_per = S // N_DEV
    n_half = n_out // 2

    Ob = O.reshape(B, S, K).astype(jnp.bfloat16)
    Wb = Wo.astype(jnp.bfloat16)

    def body(o_ref, w_ref, out_ref, comm_cw, comm_ccw, stage_cw, stage_ccw,
             x_ref, send_cw_sems, recv_cw_sems, send_ccw_sems, recv_ccw_sems,
             load_sems):
        my = lax.axis_index("i")
        left = (my + N_DEV - 1) % N_DEV
        right = (my + 1) % N_DEV

        barrier_sem = pltpu.get_barrier_semaphore()
        for nbr in (left, right):
            pl.semaphore_signal(barrier_sem, inc=1, device_id=(nbr,),
                                device_id_type=pl.DeviceIdType.MESH)
        pl.semaphore_wait(barrier_sem, 2)

        def load_chunk(c, slot):
            cp = pltpu.make_async_copy(
                o_ref.at[:, pl.ds(c * s_per, s_per), :],
                x_ref.at[slot],
                load_sems.at[slot],
            )
            cp.start()
            return cp

        def partial(slot, b, col_lo, col_n=n_half):
            return jnp.dot(x_ref[slot, b],
                           w_ref[:, pl.ds(col_lo, col_n)],
                           preferred_element_type=jnp.float32)

        dirs = [
            dict(sgn=-1, nbr=right, base=0, xbase=0, comm=comm_cw,
                 stage=stage_cw, send=send_cw_sems, recv=recv_cw_sems),
            dict(sgn=1, nbr=left, base=n_half, xbase=2, comm=comm_ccw,
                 stage=stage_ccw, send=send_ccw_sems, recv=recv_ccw_sems),
        ]

        n_sub = 2
        sub_w = n_half // n_sub

        def make_rdma(d, t, h, send_slot, recv_slot):
            sl = pl.ds(h * sub_w, sub_w)
            return pltpu.make_async_remote_copy(
                src_ref=d["comm"].at[send_slot, :, :, sl],
                dst_ref=d["comm"].at[recv_slot, :, :, sl],
                send_sem=d["send"].at[t, h],
                recv_sem=d["recv"].at[t, h],
                device_id=(d["nbr"],),
                device_id_type=pl.DeviceIdType.MESH,
            )

        for d in dirs:
            d["cp0"] = load_chunk((my + N_DEV + d["sgn"]) % N_DEV, d["xbase"])
            d["cp1"] = load_chunk((my + N_DEV + 2 * d["sgn"]) % N_DEV,
                                  d["xbase"] + 1)
        for d in dirs:
            d["cp0"].wait()
            d["rdmas"] = [None] * n_sub
            for h in range(n_sub):
                sl = pl.ds(h * sub_w, sub_w)
                for b in range(B):
                    d["comm"][0, b, :, sl] = partial(
                        d["xbase"], b, d["base"] + h * sub_w,
                        sub_w).astype(jnp.bfloat16)
                d["rdmas"][h] = make_rdma(d, 0, h, 0, 1)
                d["rdmas"][h].start()
        for d in dirs:
            d["cp1"].wait()
            for b in range(B):
                d["stage"][b] = partial(
                    d["xbase"] + 1, b, d["base"]).astype(jnp.bfloat16)

        for t in range(N_DEV - 1):
            send_slot = t % 2
            recv_slot = (t + 1) % 2
            if t < N_DEV - 2:
                for d in dirs:
                    d["cp"] = load_chunk(
                        (my + 3 * N_DEV + d["sgn"] * (t + 3)) % N_DEV,
                        d["xbase"] + t % 2)
            for h in range(n_sub):
                for d in dirs:
                    sl = pl.ds(h * sub_w, sub_w)
                    d["rdmas"][h].wait()
                    if t < N_DEV - 2:
                        d["comm"][recv_slot, :, :, sl] = (
                            d["comm"][recv_slot, :, :, sl].astype(jnp.float32)
                            + d["stage"][:, :, sl].astype(jnp.float32)
                        ).astype(jnp.bfloat16)
                        d["rdmas"][h] = make_rdma(
                            d, t + 1, h, recv_slot, send_slot)
                        d["rdmas"][h].start()
                    else:
                        out_ref[:, :, pl.ds(d["base"] + h * sub_w, sub_w)] = (
                            d["comm"][recv_slot, :, :, sl].astype(jnp.float32)
                            + d["stage"][:, :, sl].astype(jnp.float32)
                        ).astype(jnp.bfloat16)
            if t < N_DEV - 2:
                for d in dirs:
                    d["cp"].wait()
                    for b in range(B):
                        d["stage"][b] = partial(
                            d["xbase"] + t % 2, b,
                            d["base"]).astype(jnp.bfloat16)

    return pl.pallas_call(
        body,
        out_shape=jax.ShapeDtypeStruct((B, s_per, n_out), jnp.bfloat16),
        in_specs=[pl.BlockSpec(memory_space=pl.ANY),
                  pl.BlockSpec(memory_space=pltpu.VMEM)],
        out_specs=pl.BlockSpec(memory_space=pltpu.VMEM),
        scratch_shapes=[
            pltpu.VMEM((2, B, s_per, n_half), jnp.bfloat16),
            pltpu.VMEM((2, B, s_per, n_half), jnp.bfloat16),
            pltpu.VMEM((B, s_per, n_half), jnp.bfloat16),
            pltpu.VMEM((B, s_per, n_half), jnp.bfloat16),
            pltpu.VMEM((4, B, s_per, K), jnp.bfloat16),
            pltpu.SemaphoreType.DMA((N_DEV - 1, 2)),
            pltpu.SemaphoreType.DMA((N_DEV - 1, 2)),
            pltpu.SemaphoreType.DMA((N_DEV - 1, 2)),
            pltpu.SemaphoreType.DMA((N_DEV - 1, 2)),
            pltpu.SemaphoreType.DMA((4,)),
        ],
        compiler_params=pltpu.CompilerParams(
            collective_id=0,
            vmem_limit_bytes=100 * 1024 * 1024,
        ),
    )(Ob, Wb)


# device time: 355334 ns/iter; 1.9320x vs baseline; 1.0119x over previous
import jax
import jax.numpy as jnp
from jax import lax
from jax.experimental import pallas as pl
from jax.experimental.pallas import tpu as pltpu

N_DEV = 8


def kernel(O, Wo):
    B, S, H, D = O.shape
    K = H * D
    n_out = Wo.shape[1]
    s_per = S // N_DEV
    n_half = n_out // 2

    Ob = O.reshape(B, S, K).astype(jnp.bfloat16)

    def body(o_ref, wf_ref, out_ref, comm_cw, comm_ccw, stage_cw, stage_ccw,
             x_ref, w_ref, send_cw_sems, recv_cw_sems, send_ccw_sems,
             recv_ccw_sems, load_sems):
        my = lax.axis_index("i")
        left = (my + N_DEV - 1) % N_DEV
        right = (my + 1) % N_DEV

        barrier_sem = pltpu.get_barrier_semaphore()
        for nbr in (left, right):
            pl.semaphore_signal(barrier_sem, inc=1, device_id=(nbr,),
                                device_id_type=pl.DeviceIdType.MESH)
        pl.semaphore_wait(barrier_sem, 2)

        def load_chunk(c, slot):
            cp = pltpu.make_async_copy(
                o_ref.at[:, pl.ds(c * s_per, s_per), :],
                x_ref.at[slot],
                load_sems.at[slot],
            )
            cp.start()
            return cp

        def partial(slot, b, col_lo, col_n=n_half):
            return jnp.dot(x_ref[slot, b],
                           w_ref[:, pl.ds(col_lo, col_n)],
                           preferred_element_type=jnp.float32)

        dirs = [
            dict(sgn=-1, nbr=right, base=0, xbase=0, comm=comm_cw,
                 stage=stage_cw, send=send_cw_sems, recv=recv_cw_sems),
            dict(sgn=1, nbr=left, base=n_half, xbase=2, comm=comm_ccw,
                 stage=stage_ccw, send=send_ccw_sems, recv=recv_ccw_sems),
        ]

        n_sub = 4
        sub_w = n_half // n_sub

        def make_rdma(d, t, h, send_slot, recv_slot):
            sl = pl.ds(h * sub_w, sub_w)
            return pltpu.make_async_remote_copy(
                src_ref=d["comm"].at[send_slot, :, :, sl],
                dst_ref=d["comm"].at[recv_slot, :, :, sl],
                send_sem=d["send"].at[t, h],
                recv_sem=d["recv"].at[t, h],
                device_id=(d["nbr"],),
                device_id_type=pl.DeviceIdType.MESH,
            )

        for d in dirs:
            d["cp0"] = load_chunk((my + N_DEV + d["sgn"]) % N_DEV, d["xbase"])
            d["cp1"] = load_chunk((my + N_DEV + 2 * d["sgn"]) % N_DEV,
                                  d["xbase"] + 1)
        w_ref[...] = wf_ref[...].astype(jnp.bfloat16)
        for d in dirs:
            d["cp0"].wait()
            d["rdmas"] = [None] * n_sub
            for h in range(n_sub):
                sl = pl.ds(h * sub_w, sub_w)
                for b in range(B):
                    d["comm"][0, b, :, sl] = partial(
                        d["xbase"], b, d["base"] + h * sub_w,
                        sub_w).astype(jnp.bfloat16)
                d["rdmas"][h] = make_rdma(d, 0, h, 0, 1)
                d["rdmas"][h].start()
        for d in dirs:
            d["cp1"].wait()
            for b in range(B):
                d["stage"][b] = partial(
                    d["xbase"] + 1, b, d["base"]).astype(jnp.bfloat16)

        for t in range(N_DEV - 1):
            send_slot = t % 2
            recv_slot = (t + 1) % 2
            if t < N_DEV - 2:
                for d in dirs:
                    d["cp"] = load_chunk(
                        (my + 3 * N_DEV + d["sgn"] * (t + 3)) % N_DEV,
                        d["xbase"] + t % 2)
            for h in range(n_sub):
                for d in dirs:
                    sl = pl.ds(h * sub_w, sub_w)
                    d["rdmas"][h].wait()
                    if t < N_DEV - 2:
                        d["comm"][recv_slot, :, :, sl] = (
                            d["comm"][recv_slot, :, :, sl].astype(jnp.float32)
                            + d["stage"][:, :, sl].astype(jnp.float32)
                        ).astype(jnp.bfloat16)
                        d["rdmas"][h] = make_rdma(
                            d, t + 1, h, recv_slot, send_slot)
                        d["rdmas"][h].start()
                    else:
                        out_ref[:, :, pl.ds(d["base"] + h * sub_w, sub_w)] = (
                            d["comm"][recv_slot, :, :, sl].astype(jnp.float32)
                            + d["stage"][:, :, sl].astype(jnp.float32)
                        ).astype(jnp.bfloat16)
            if t < N_DEV - 2:
                for d in dirs:
                    d["cp"].wait()
                    for b in range(B):
                        d["stage"][b] = partial(
                            d["xbase"] + t % 2, b,
                            d["base"]).astype(jnp.bfloat16)

    return pl.pallas_call(
        body,
        out_shape=jax.ShapeDtypeStruct((B, s_per, n_out), jnp.bfloat16),
        in_specs=[pl.BlockSpec(memory_space=pl.ANY),
                  pl.BlockSpec(memory_space=pltpu.VMEM)],
        out_specs=pl.BlockSpec(memory_space=pltpu.VMEM),
        scratch_shapes=[
            pltpu.VMEM((2, B, s_per, n_half), jnp.bfloat16),
            pltpu.VMEM((2, B, s_per, n_half), jnp.bfloat16),
            pltpu.VMEM((B, s_per, n_half), jnp.bfloat16),
            pltpu.VMEM((B, s_per, n_half), jnp.bfloat16),
            pltpu.VMEM((4, B, s_per, K), jnp.bfloat16),
            pltpu.VMEM((K, n_out), jnp.bfloat16),
            pltpu.SemaphoreType.DMA((N_DEV - 1, 4)),
            pltpu.SemaphoreType.DMA((N_DEV - 1, 4)),
            pltpu.SemaphoreType.DMA((N_DEV - 1, 4)),
            pltpu.SemaphoreType.DMA((N_DEV - 1, 4)),
            pltpu.SemaphoreType.DMA((4,)),
        ],
        compiler_params=pltpu.CompilerParams(
            collective_id=0,
            vmem_limit_bytes=100 * 1024 * 1024,
        ),
    )(Ob, Wo)
